# baseline (speedup 1.0000x reference)
import functools

import jax
import jax.numpy as jnp
from jax import lax
from jax.experimental import pallas as pl
from jax.experimental.pallas import tpu as pltpu
from jax.experimental.pallas import tpu_sc as plsc

N = 10000
D = 128
E = 320000

NC = 2
NS = 16
NW = NC * NS

RPS = 128
K_INNER = 8
E_PER_W = 10240
E_PAD = NW * E_PER_W
N_OUTER = E_PER_W // (RPS * K_INNER)
IDXROWS_PER_W = E_PER_W // RPS

ZROWS = 632
N_ACC = NS * ZROWS

_mesh = plsc.VectorSubcoreMesh(core_axis_name="c", subcore_axis_name="s")


NBUF = 2
SUB = 2
T_PER_W = E_PER_W // RPS
M_ITERS = T_PER_W // NBUF
ROWS_C0 = 112
ROWS_PAIR = 2 * IDXROWS_PER_W


def _agg_body(x_hbm, src_hbm, dst_hbm, zrow_hbm, out_hbm,
              src_v, dst_v, rows_v, acc_sh, *sems):
    semg = sems[:NBUF]
    sems_s = sems[NBUF:]
    c = lax.axis_index("c")
    s = lax.axis_index("s")
    wid = s * NC + c

    pltpu.sync_copy(zrow_hbm, acc_sh.at[pl.ds(s * ZROWS, ZROWS)])
    plsc.subcore_barrier()

    row0 = s * ROWS_PAIR + jnp.where(c == 0, 0, ROWS_C0)
    t_rows = jnp.where(c == 0, ROWS_C0, ROWS_PAIR - ROWS_C0)
    miters = t_rows // NBUF
    hbm_dummy = x_hbm.at[pl.ds(0, RPS)]
    SR = RPS // SUB

    def _drain(sem, buf):
        pltpu.make_async_copy(hbm_dummy, buf, sem).wait()

    def _gather(b, rk):
        for q in range(SUB):
            pltpu.async_copy(
                x_hbm.at[src_v.at[rk, pl.ds(q * SR, SR)]],
                rows_v.at[b, pl.ds(q * SR, SR)], semg[b])

    def _scatter(b, rk):
        pltpu.async_copy(rows_v.at[b], acc_sh.at[dst_v.at[rk]],
                         sems_s[b], add=True)

    def _outer(m, carry):
        for k in range(NBUF):
            rk_g = 2 * jnp.remainder(m, 8) + k
            rk_s = jnp.remainder(rk_g + 15, 16)
            b = k
            b2 = 1 - k
            if k == 0:
                @pl.when(jnp.remainder(m, 4) == 0)
                def _reload():
                    r = row0 + (m // 4) * 8
                    h = jnp.remainder(m // 4, 2) * 8
                    pltpu.sync_copy(src_hbm.at[pl.ds(r, 8)],
                                    src_v.at[pl.ds(h, 8)])
                    pltpu.sync_copy(dst_hbm.at[pl.ds(r, 8)],
                                    dst_v.at[pl.ds(h, 8)])

            @pl.when(m > 0)
            def _guard():
                _drain(sems_s[b], rows_v.at[b])
            _gather(b, rk_g)

            def _retire():
                _drain(semg[b2], rows_v.at[b2])
                _scatter(b2, rk_s)
            if k == 0:
                pl.when(m > 0)(_retire)
            else:
                _retire()
        return carry

    lax.fori_loop(0, miters, _outer, 0)

    _drain(semg[1], rows_v.at[1])
    _scatter(1, jnp.remainder(t_rows - 1, 16))
    for b in range(NBUF):
        _drain(sems_s[b], rows_v.at[b])

    plsc.subcore_barrier()

    pltpu.sync_copy(acc_sh.at[pl.ds(s * ZROWS, ZROWS)],
                    out_hbm.at[c, pl.ds(s * ZROWS, ZROWS)])


_agg = pl.kernel(
    _agg_body,
    out_type=jax.ShapeDtypeStruct((NC, N_ACC, D), jnp.float32),
    mesh=_mesh,
    scratch_types=[
        pltpu.VMEM((16, RPS), jnp.int32),
        pltpu.VMEM((16, RPS), jnp.int32),
        pltpu.VMEM((NBUF, RPS, D), jnp.float32),
        pltpu.VMEM_SHARED((N_ACC, D), jnp.float32),
    ] + [pltpu.SemaphoreType.DMA] * (2 * NBUF),
)


def _cnt_body(dst_hbm, zrow_hbm, ones_hbm, out_hbm,
              dst_v, ones_v, acc_sh, sem_a, sem_b):
    c = lax.axis_index("c")
    s = lax.axis_index("s")
    wid = s * NC + c

    pltpu.sync_copy(zrow_hbm, acc_sh.at[pl.ds(s * ZROWS, ZROWS)])
    pltpu.sync_copy(ones_hbm, ones_v)
    plsc.subcore_barrier()

    row0 = wid * IDXROWS_PER_W
    hbm_dummy = zrow_hbm.at[pl.ds(0, RPS)]
    half_sems = (sem_a, sem_b)

    def _outer(mm, carry):
        for h in range(2):
            m = 2 * mm + h
            sem = half_sems[h]

            @pl.when(mm > 0)
            def _dr():
                for _ in range(K_INNER):
                    pltpu.make_async_copy(hbm_dummy, ones_v, sem).wait()
            r = row0 + m * K_INNER
            pltpu.sync_copy(dst_hbm.at[pl.ds(r, K_INNER)],
                            dst_v.at[pl.ds(h * K_INNER, K_INNER)])
            for j in range(K_INNER):
                pltpu.async_copy(ones_v, acc_sh.at[dst_v.at[h * K_INNER + j]],
                                 sem, add=True)
        return carry

    lax.fori_loop(0, N_OUTER // 2, _outer, 0)
    for sem in half_sems:
        for _ in range(K_INNER):
            pltpu.make_async_copy(hbm_dummy, ones_v, sem).wait()

    plsc.subcore_barrier()

    pltpu.sync_copy(acc_sh.at[pl.ds(s * ZROWS, ZROWS)],
                    out_hbm.at[c, pl.ds(s * ZROWS, ZROWS)])


_cnt = pl.kernel(
    _cnt_body,
    out_type=jax.ShapeDtypeStruct((NC, N_ACC, D), jnp.float32),
    mesh=_mesh,
    scratch_types=[
        pltpu.VMEM((2 * K_INNER, RPS), jnp.int32),
        pltpu.VMEM((RPS, D), jnp.float32),
        pltpu.VMEM_SHARED((N_ACC, D), jnp.float32),
        pltpu.SemaphoreType.DMA,
        pltpu.SemaphoreType.DMA,
    ],
)


BLK = 2000


def _dense_body(relu, s0, s1, c0, c1, x, wl, wr, b, o):
    cnt = c0[0, :, :1] + c1[0, :, :1]
    mean = (s0[0] + s1[0]) / jnp.maximum(cnt, 1.0)
    h = (jnp.dot(mean, wl[...], preferred_element_type=jnp.float32)
         + jnp.dot(x[...], wr[...], preferred_element_type=jnp.float32)
         + b[...])
    o[...] = jnp.maximum(h, 0.0) if relu else h


def _dense(relu, sums, cnts, x, wlT, wrT, b):
    grid = N // BLK
    s_blk0 = pl.BlockSpec((1, BLK, D), lambda i: (0, i, 0))
    s_blk1 = pl.BlockSpec((1, BLK, D), lambda i: (1, i, 0))
    row_blk = pl.BlockSpec((BLK, D), lambda i: (i, 0))
    full = pl.BlockSpec((D, D), lambda i: (0, 0))
    bias = pl.BlockSpec((1, D), lambda i: (0, 0))
    return pl.pallas_call(
        functools.partial(_dense_body, relu),
        grid=(grid,),
        in_specs=[s_blk0, s_blk1, s_blk0, s_blk1, row_blk, full, full, bias],
        out_specs=row_blk,
        out_shape=jax.ShapeDtypeStruct((N, D), jnp.float32),
    )(sums, sums, cnts, cnts, x, wlT, wrT, b)


def kernel(x, edge_index, Wl1, bl1, Wr1, Wl2, bl2, Wr2):
    src = edge_index[0].astype(jnp.int32)
    dst = edge_index[1].astype(jnp.int32)
    pad = E_PAD - E
    src_p = jnp.concatenate([src, jnp.zeros((pad,), jnp.int32)])
    dst_p = jnp.concatenate([dst, jnp.full((pad,), N, jnp.int32)])
    src_p = src_p.reshape(E_PAD // RPS, RPS)
    dst_p = dst_p.reshape(E_PAD // RPS, RPS)
    zrow = jnp.zeros((ZROWS, D), jnp.float32)
    ones = jnp.ones((RPS, D), jnp.float32)

    cnts = _cnt(dst_p, zrow, ones)
    sums1 = _agg(x, src_p, dst_p, zrow)
    h = _dense(True, sums1, cnts, x, Wl1.T, Wr1.T, bl1.reshape(1, D))
    sums2 = _agg(h, src_p, dst_p, zrow)
    out = _dense(False, sums2, cnts, h, Wl2.T, Wr2.T, bl2.reshape(1, D))
    return out


# speedup vs baseline: 1.0456x; 1.0456x over previous
import functools

import jax
import jax.numpy as jnp
from jax import lax
from jax.experimental import pallas as pl
from jax.experimental.pallas import tpu as pltpu
from jax.experimental.pallas import tpu_sc as plsc

N = 10000
D = 128
E = 320000

NC = 2
NS = 16
NW = NC * NS

RPS = 128
K_INNER = 8
E_PER_W = 10240
E_PAD = NW * E_PER_W
N_OUTER = E_PER_W // (RPS * K_INNER)
IDXROWS_PER_W = E_PER_W // RPS

ZROWS = 632
N_ACC = NS * ZROWS

_mesh = plsc.VectorSubcoreMesh(core_axis_name="c", subcore_axis_name="s")


NBUF = 2
SUB = 2
T_PER_W = E_PER_W // RPS
M_ITERS = T_PER_W // NBUF
ROWS_C0 = 128
ROWS_PAIR = 2 * IDXROWS_PER_W


def _agg_body(x_hbm, src_hbm, dst_hbm, zrow_hbm, out_hbm,
              src_v, dst_v, rows_v, acc_sh, *sems):
    semg = sems[:NBUF]
    sems_s = sems[NBUF:]
    c = lax.axis_index("c")
    s = lax.axis_index("s")
    wid = s * NC + c

    pltpu.sync_copy(zrow_hbm, acc_sh.at[pl.ds(s * ZROWS, ZROWS)])
    plsc.subcore_barrier()

    row0 = s * ROWS_PAIR + jnp.where(c == 0, 0, ROWS_C0)
    t_rows = jnp.where(c == 0, ROWS_C0, ROWS_PAIR - ROWS_C0)
    miters = t_rows // NBUF
    hbm_dummy = x_hbm.at[pl.ds(0, RPS)]
    SR = RPS // SUB

    def _drain(sem, buf):
        pltpu.make_async_copy(hbm_dummy, buf, sem).wait()

    def _gather(b, rk):
        for q in range(SUB):
            pltpu.async_copy(
                x_hbm.at[src_v.at[rk, pl.ds(q * SR, SR)]],
                rows_v.at[b, pl.ds(q * SR, SR)], semg[b])

    def _scatter(b, rk):
        pltpu.async_copy(rows_v.at[b], acc_sh.at[dst_v.at[rk]],
                         sems_s[b], add=True)

    def _outer(m, carry):
        for k in range(NBUF):
            rk_g = 2 * jnp.remainder(m, 8) + k
            rk_s = jnp.remainder(rk_g + 15, 16)
            b = k
            b2 = 1 - k
            if k == 0:
                @pl.when(jnp.remainder(m, 4) == 0)
                def _reload():
                    r = row0 + (m // 4) * 8
                    h = jnp.remainder(m // 4, 2) * 8
                    pltpu.sync_copy(src_hbm.at[pl.ds(r, 8)],
                                    src_v.at[pl.ds(h, 8)])
                    pltpu.sync_copy(dst_hbm.at[pl.ds(r, 8)],
                                    dst_v.at[pl.ds(h, 8)])

            @pl.when(m > 0)
            def _guard():
                _drain(sems_s[b], rows_v.at[b])
            _gather(b, rk_g)

            def _retire():
                _drain(semg[b2], rows_v.at[b2])
                _scatter(b2, rk_s)
            if k == 0:
                pl.when(m > 0)(_retire)
            else:
                _retire()
        return carry

    lax.fori_loop(0, miters, _outer, 0)

    _drain(semg[1], rows_v.at[1])
    _scatter(1, jnp.remainder(t_rows - 1, 16))
    for b in range(NBUF):
        _drain(sems_s[b], rows_v.at[b])

    plsc.subcore_barrier()

    pltpu.sync_copy(acc_sh.at[pl.ds(s * ZROWS, ZROWS)],
                    out_hbm.at[c, pl.ds(s * ZROWS, ZROWS)])


_agg = pl.kernel(
    _agg_body,
    out_type=jax.ShapeDtypeStruct((NC, N_ACC, D), jnp.float32),
    mesh=_mesh,
    scratch_types=[
        pltpu.VMEM((16, RPS), jnp.int32),
        pltpu.VMEM((16, RPS), jnp.int32),
        pltpu.VMEM((NBUF, RPS, D), jnp.float32),
        pltpu.VMEM_SHARED((N_ACC, D), jnp.float32),
    ] + [pltpu.SemaphoreType.DMA] * (2 * NBUF),
)


def _cnt_body(dst_hbm, zrow_hbm, ones_hbm, out_hbm,
              dst_v, ones_v, acc_sh, sem_a, sem_b):
    c = lax.axis_index("c")
    s = lax.axis_index("s")
    wid = s * NC + c

    pltpu.sync_copy(zrow_hbm, acc_sh.at[pl.ds(s * ZROWS, ZROWS)])
    pltpu.sync_copy(ones_hbm, ones_v)
    plsc.subcore_barrier()

    row0 = wid * IDXROWS_PER_W
    hbm_dummy = zrow_hbm.at[pl.ds(0, RPS)]
    half_sems = (sem_a, sem_b)

    def _outer(mm, carry):
        for h in range(2):
            m = 2 * mm + h
            sem = half_sems[h]

            @pl.when(mm > 0)
            def _dr():
                for _ in range(K_INNER):
                    pltpu.make_async_copy(hbm_dummy, ones_v, sem).wait()
            r = row0 + m * K_INNER
            pltpu.sync_copy(dst_hbm.at[pl.ds(r, K_INNER)],
                            dst_v.at[pl.ds(h * K_INNER, K_INNER)])
            for j in range(K_INNER):
                pltpu.async_copy(ones_v, acc_sh.at[dst_v.at[h * K_INNER + j]],
                                 sem, add=True)
        return carry

    lax.fori_loop(0, N_OUTER // 2, _outer, 0)
    for sem in half_sems:
        for _ in range(K_INNER):
            pltpu.make_async_copy(hbm_dummy, ones_v, sem).wait()

    plsc.subcore_barrier()

    pltpu.sync_copy(acc_sh.at[pl.ds(s * ZROWS, ZROWS)],
                    out_hbm.at[c, pl.ds(s * ZROWS, ZROWS)])


_cnt = pl.kernel(
    _cnt_body,
    out_type=jax.ShapeDtypeStruct((NC, N_ACC, D), jnp.float32),
    mesh=_mesh,
    scratch_types=[
        pltpu.VMEM((2 * K_INNER, RPS), jnp.int32),
        pltpu.VMEM((RPS, D), jnp.float32),
        pltpu.VMEM_SHARED((N_ACC, D), jnp.float32),
        pltpu.SemaphoreType.DMA,
        pltpu.SemaphoreType.DMA,
    ],
)


BLK = 2000


def _dense_body(relu, s0, s1, c0, c1, x, wl, wr, b, o):
    cnt = c0[0, :, :1] + c1[0, :, :1]
    mean = (s0[0] + s1[0]) / jnp.maximum(cnt, 1.0)
    h = (jnp.dot(mean, wl[...], preferred_element_type=jnp.float32)
         + jnp.dot(x[...], wr[...], preferred_element_type=jnp.float32)
         + b[...])
    o[...] = jnp.maximum(h, 0.0) if relu else h


def _dense(relu, sums, cnts, x, wlT, wrT, b):
    grid = N // BLK
    s_blk0 = pl.BlockSpec((1, BLK, D), lambda i: (0, i, 0))
    s_blk1 = pl.BlockSpec((1, BLK, D), lambda i: (1, i, 0))
    row_blk = pl.BlockSpec((BLK, D), lambda i: (i, 0))
    full = pl.BlockSpec((D, D), lambda i: (0, 0))
    bias = pl.BlockSpec((1, D), lambda i: (0, 0))
    return pl.pallas_call(
        functools.partial(_dense_body, relu),
        grid=(grid,),
        in_specs=[s_blk0, s_blk1, s_blk0, s_blk1, row_blk, full, full, bias],
        out_specs=row_blk,
        out_shape=jax.ShapeDtypeStruct((N, D), jnp.float32),
    )(sums, sums, cnts, cnts, x, wlT, wrT, b)


def kernel(x, edge_index, Wl1, bl1, Wr1, Wl2, bl2, Wr2):
    src = edge_index[0].astype(jnp.int32)
    dst = edge_index[1].astype(jnp.int32)
    pad = E_PAD - E
    src_p = jnp.concatenate([src, jnp.zeros((pad,), jnp.int32)])
    dst_p = jnp.concatenate([dst, jnp.full((pad,), N, jnp.int32)])
    src_p = src_p.reshape(E_PAD // RPS, RPS)
    dst_p = dst_p.reshape(E_PAD // RPS, RPS)
    zrow = jnp.zeros((ZROWS, D), jnp.float32)
    ones = jnp.ones((RPS, D), jnp.float32)

    cnts = _cnt(dst_p, zrow, ones)
    sums1 = _agg(x, src_p, dst_p, zrow)
    h = _dense(True, sums1, cnts, x, Wl1.T, Wr1.T, bl1.reshape(1, D))
    sums2 = _agg(h, src_p, dst_p, zrow)
    out = _dense(False, sums2, cnts, h, Wl2.T, Wr2.T, bl2.reshape(1, D))
    return out
